# baseline (speedup 1.0000x reference)
import functools

import jax
import jax.numpy as jnp
from jax import lax
from jax.experimental import pallas as pl
from jax.experimental.pallas import tpu as pltpu
from jax.experimental.pallas import tpu_sc as plsc

P = 3
M = 512
S = 128
OUT_DIM = 512
DIM = 3
N_KNOTS = M + P + 1
N_INT = N_KNOTS - 2 * P
KNOT_PAD = 768
L = 16
NW = 32
CURVES_PER_W = S // NW
VREGS = OUT_DIM // L


def _spline_body(knot_hbm, ctrl_hbm, u_hbm, out_hbm, knot_v, ctrl_v, u_v, out_v,
                 sem_k, sem_c):
    wid = lax.axis_index("s") * 2 + lax.axis_index("c")
    base = wid * CURVES_PER_W
    iota = lax.iota(jnp.int32, L)

    kh = pltpu.async_copy(knot_hbm.at[pl.ds(base, CURVES_PER_W)], knot_v, sem_k)
    ch = pltpu.async_copy(ctrl_hbm.at[pl.ds(base, CURVES_PER_W)], ctrl_v, sem_c)
    pltpu.sync_copy(u_hbm, u_v)
    kh.wait()
    ch.wait()

    @plsc.parallel_loop(0, CURVES_PER_W * OUT_DIM, step=L, unroll=1)
    def _vloop(g0):
        cc = jnp.zeros((L,), jnp.int32) + (g0 >> 9)
        j0 = g0 & (OUT_DIM - 1)
        ju = j0 + iota
        u = u_v[pl.ds(j0, L)]

        pos = jnp.zeros((L,), jnp.int32)
        for b in (256, 128, 64, 32, 16, 8, 4, 2, 1):
            cand = pos + b
            g = plsc.load_gather(knot_v, [cc, cand + (P - 1)])
            pos = jnp.where((u - g) > 1e-8, cand, pos)
        gl = plsc.load_gather(knot_v, [cc, P + pos - 1])
        dmin = u - gl

        pos2 = jnp.zeros((L,), jnp.int32)
        for b in (256, 128, 64, 32, 16, 8, 4, 2, 1):
            cand = pos2 + b
            g = plsc.load_gather(knot_v, [cc, cand + (P - 1)])
            pos2 = jnp.where((u - g) > dmin, cand, pos2)
        uspan = pos2 + P

        Ug = {}
        for d in range(-2, 4):
            Ug[d] = plsc.load_gather(knot_v, [cc, uspan + d])

        Ni = [None] * (P + 1)
        Ni[0] = jnp.ones((L,), jnp.float32)
        for k in range(1, P + 1):
            saved = jnp.zeros((L,), jnp.float32)
            for r in range(k):
                U1 = Ug[r + 1]
                U2 = Ug[1 - k + r]
                den = (U1 - u) + (u - U2)
                zero = den == 0.0
                safe_den = jnp.where(zero, 1.0, den)
                temp = Ni[r] / safe_den
                temp = jnp.where(zero, 0.0001, temp)
                Ni[r] = saved + (U1 - u) * temp
                saved = (u - U2) * temp
            Ni[k] = saved

        cbase = uspan * DIM - P * DIM
        for d in range(DIM):
            acc = jnp.zeros((L,), jnp.float32)
            for r in range(P + 1):
                pts = plsc.load_gather(ctrl_v, [cc, cbase + (r * DIM + d)])
                acc = acc + Ni[r] * pts
            plsc.store_scatter(out_v, [cc, ju * DIM + d], acc)

    pltpu.sync_copy(out_v, out_hbm.at[pl.ds(base, CURVES_PER_W)])


@jax.jit
def _curve_eval_sc(knot_pad, ctrl_flat, u):
    mesh = plsc.VectorSubcoreMesh(
        core_axis_name="c", subcore_axis_name="s", num_cores=2,
        num_subcores=16)
    run = functools.partial(
        pl.kernel,
        mesh=mesh,
        compiler_params=pltpu.CompilerParams(needs_layout_passes=False),
        out_type=jax.ShapeDtypeStruct((S, OUT_DIM * DIM), jnp.float32),
        scratch_types=[
            pltpu.VMEM((CURVES_PER_W, KNOT_PAD), jnp.float32),
            pltpu.VMEM((CURVES_PER_W, M * DIM), jnp.float32),
            pltpu.VMEM((OUT_DIM,), jnp.float32),
            pltpu.VMEM((CURVES_PER_W, OUT_DIM * DIM), jnp.float32),
            pltpu.SemaphoreType.DMA,
            pltpu.SemaphoreType.DMA,
        ],
    )(_spline_body)
    return run(knot_pad, ctrl_flat, u)


def kernel(ctrl_pts, knot_u):
    u = jnp.linspace(1e-05, 1.0 - 1e-05, OUT_DIM, dtype=jnp.float32)
    knot_pad = jnp.concatenate(
        [knot_u, jnp.full((S, KNOT_PAD - N_KNOTS), 2.0, jnp.float32)], axis=1)
    ctrl_flat = ctrl_pts.reshape(S, M * DIM)
    out = _curve_eval_sc(knot_pad, ctrl_flat, u)
    return out.reshape(S, OUT_DIM, DIM)


# speedup vs baseline: 1.0584x; 1.0584x over previous
import functools

import jax
import jax.numpy as jnp
from jax import lax
from jax.experimental import pallas as pl
from jax.experimental.pallas import tpu as pltpu
from jax.experimental.pallas import tpu_sc as plsc

P = 3
M = 512
S = 128
OUT_DIM = 512
DIM = 3
N_KNOTS = M + P + 1
N_INT = N_KNOTS - 2 * P
KNOT_PAD = 768
L = 16
NW = 32
CURVES_PER_W = S // NW
VREGS = OUT_DIM // L


def _spline_body(knot_hbm, ctrl_hbm, u_hbm, out_hbm, knot_v, ctrl_v, u_v, out_v,
                 sem_k, sem_c, sem_o):
    wid = lax.axis_index("s") * 2 + lax.axis_index("c")
    pltpu.sync_copy(u_hbm, u_v)
    iota = lax.iota(jnp.int32, L)

    def eval_one(j0):
        ju = j0 + iota
        u = u_v[pl.ds(j0, L)]

        pos = jnp.zeros((L,), jnp.int32)
        for b in (256, 128, 64, 32, 16, 8, 4, 2, 1):
            cand = pos + b
            g = plsc.load_gather(knot_v, [cand + (P - 1)])
            pos = jnp.where((u - g) > 1e-8, cand, pos)
        gl = plsc.load_gather(knot_v, [P + pos - 1])
        dmin = u - gl

        pos2 = jnp.zeros((L,), jnp.int32)
        for b in (256, 128, 64, 32, 16, 8, 4, 2, 1):
            cand = pos2 + b
            g = plsc.load_gather(knot_v, [cand + (P - 1)])
            pos2 = jnp.where((u - g) > dmin, cand, pos2)
        uspan = pos2 + P

        Ug = {}
        for d in range(-2, 4):
            Ug[d] = plsc.load_gather(knot_v, [uspan + d])

        Ni = [None] * (P + 1)
        Ni[0] = jnp.ones((L,), jnp.float32)
        for k in range(1, P + 1):
            saved = jnp.zeros((L,), jnp.float32)
            for r in range(k):
                U1 = Ug[r + 1]
                U2 = Ug[1 - k + r]
                den = (U1 - u) + (u - U2)
                zero = den == 0.0
                safe_den = jnp.where(zero, 1.0, den)
                temp = Ni[r] / safe_den
                temp = jnp.where(zero, 0.0001, temp)
                Ni[r] = saved + (U1 - u) * temp
                saved = (u - U2) * temp
            Ni[k] = saved

        cbase = uspan * DIM - P * DIM
        for d in range(DIM):
            acc = jnp.zeros((L,), jnp.float32)
            for r in range(P + 1):
                pts = plsc.load_gather(ctrl_v, [cbase + (r * DIM + d)])
                acc = acc + Ni[r] * pts
            plsc.store_scatter(out_v, [ju * DIM + d], acc)

    def curve_body(cc, _):
        c = wid * CURVES_PER_W + cc
        kh = pltpu.async_copy(knot_hbm.at[c], knot_v, sem_k)
        ch = pltpu.async_copy(ctrl_hbm.at[c], ctrl_v, sem_c)

        @pl.when(cc > 0)
        def _():
            pltpu.make_async_copy(out_v, out_hbm.at[c - 1], sem_o).wait()

        kh.wait()
        ch.wait()

        @plsc.parallel_loop(0, OUT_DIM, step=L, unroll=1)
        def _vloop(j0):
            eval_one(j0)

        pltpu.async_copy(out_v, out_hbm.at[c], sem_o)
        return 0

    lax.fori_loop(0, CURVES_PER_W, curve_body, 0)
    last = wid * CURVES_PER_W + (CURVES_PER_W - 1)
    pltpu.make_async_copy(out_v, out_hbm.at[last], sem_o).wait()


@jax.jit
def _curve_eval_sc(knot_pad, ctrl_flat, u):
    mesh = plsc.VectorSubcoreMesh(
        core_axis_name="c", subcore_axis_name="s", num_cores=2,
        num_subcores=16)
    run = functools.partial(
        pl.kernel,
        mesh=mesh,
        compiler_params=pltpu.CompilerParams(needs_layout_passes=False),
        out_type=jax.ShapeDtypeStruct((S, OUT_DIM * DIM), jnp.float32),
        scratch_types=[
            pltpu.VMEM((KNOT_PAD,), jnp.float32),
            pltpu.VMEM((M * DIM,), jnp.float32),
            pltpu.VMEM((OUT_DIM,), jnp.float32),
            pltpu.VMEM((OUT_DIM * DIM,), jnp.float32),
            pltpu.SemaphoreType.DMA,
            pltpu.SemaphoreType.DMA,
            pltpu.SemaphoreType.DMA,
        ],
    )(_spline_body)
    return run(knot_pad, ctrl_flat, u)


def kernel(ctrl_pts, knot_u):
    u = jnp.linspace(1e-05, 1.0 - 1e-05, OUT_DIM, dtype=jnp.float32)
    knot_pad = jnp.concatenate(
        [knot_u, jnp.full((S, KNOT_PAD - N_KNOTS), 2.0, jnp.float32)], axis=1)
    ctrl_flat = ctrl_pts.reshape(S, M * DIM)
    out = _curve_eval_sc(knot_pad, ctrl_flat, u)
    return out.reshape(S, OUT_DIM, DIM)
